# baseline (speedup 1.0000x reference)
import functools

import jax
import jax.numpy as jnp
from jax import lax
from jax.experimental import pallas as pl
from jax.experimental.pallas import tpu as pltpu
from jax.experimental.pallas import tpu_sc as plsc

NSAMP = 1000000
LANES = 16
NTILES = 32
PER_TILE = 64000
PAD_SIDE = 16 * PER_TILE - NSAMP
CHUNK = 8000
NCHUNK = PER_TILE // CHUNK
NVREG = CHUNK // LANES
B = 32768


def _reduce_body(xk, yk, xw, yw, vmin_o, vmax_o, wx_o, wy_o):
    vmin_o[...] = jnp.minimum(jnp.min(xk[...]), jnp.min(yk[...])).reshape(1, 1)
    vmax_o[...] = jnp.maximum(jnp.max(xk[...]), jnp.max(yk[...])).reshape(1, 1)
    wx_o[...] = jnp.sum(xw[...]).reshape(1, 1)
    wy_o[...] = jnp.sum(yw[...]).reshape(1, 1)


def _reduce_tc(xk, yk, xw, yw):
    s11 = jax.ShapeDtypeStruct((1, 1), jnp.float32)
    return pl.pallas_call(
        _reduce_body,
        out_shape=(s11, s11, s11, s11),
    )(xk, yk, xw, yw)


def _sc_hist_body(keys_hbm, w_hbm, consts_hbm, out_hbm,
                  wsum, iwsum, cvm, kbuf0, kbuf1, wbuf0, wbuf1,
                  ksem, wsem):
    kbufs = (kbuf0, kbuf1)
    wbufs = (wbuf0, wbuf1)
    cid = lax.axis_index("c")
    sid = lax.axis_index("s")
    wid = sid * 2 + cid
    base = wid * PER_TILE

    pltpu.sync_copy(consts_hbm.at[wid], cvm)
    vmin_v = cvm[0, :]
    h_v = cvm[1, :]
    inv_h_v = cvm[2, :]
    c_v = cvm[3, :]

    lane = lax.iota(jnp.int32, 16)
    lanem1 = jnp.maximum(lane - 1, 0)
    lanep1 = jnp.minimum(lane + 1, 15)
    zeros16 = jnp.zeros((16,), jnp.float32)

    def zero_body(i, _):
        for u in range(8):
            wsum[pl.ds(i * 128 + u * 16, 16)] = zeros16
            iwsum[pl.ds(i * 128 + u * 16, 16)] = zeros16
        return 0

    lax.fori_loop(0, B // 128, zero_body, 0)

    def start_dma(c, slot):
        st = base + c * CHUNK
        ck = pltpu.async_copy(keys_hbm.at[pl.ds(st, CHUNK)],
                              kbufs[slot], ksem.at[slot])
        cw = pltpu.async_copy(w_hbm.at[pl.ds(st, CHUNK)],
                              wbufs[slot], wsem.at[slot])
        return ck, cw

    pend = {0: start_dma(0, 0)}
    for c in range(NCHUNK):
        slot = c % 2
        if c + 1 < NCHUNK:
            pend[(c + 1) % 2] = start_dma(c + 1, (c + 1) % 2)
        ck, cw = pend[slot]
        ck.wait()
        cw.wait()

        kb = kbufs[slot]
        wb = wbufs[slot]

        def body(i, _):
            for u in range(4):
                k = kb[pl.ds(i * 64 + u * 16, 16)]
                w = wb[pl.ds(i * 64 + u * 16, 16)]
                t = (k - vmin_v) * inv_h_v
                b = jnp.clip(t.astype(jnp.int32), 0, B - 1)
                s = w * c_v
                dd = (b.astype(jnp.float32) + 1.0) * h_v + vmin_v - k
                d = jnp.clip(dd, 0.0, h_v)
                ids, ss = plsc.sort_key_val(b, s)
                _, sds = plsc.sort_key_val(b, s * d)
                prev = ids.at[lanem1].get(mode="promise_in_bounds")
                nxt = ids.at[lanep1].get(mode="promise_in_bounds")
                newg = (lane == 0) | (ids != prev)
                is_last = (lane == 15) | (ids != nxt)
                gstart = plsc.cummax(jnp.where(newg, lane, 0))
                ps = plsc.cumsum(ss)
                ps2 = plsc.cumsum(sds)
                gm1 = jnp.maximum(gstart - 1, 0)
                first = gstart == 0
                bps = jnp.where(first, 0.0,
                                ps.at[gm1].get(mode="promise_in_bounds"))
                bps2 = jnp.where(first, 0.0,
                                 ps2.at[gm1].get(mode="promise_in_bounds"))
                plsc.addupdate_scatter(wsum, [ids], ps - bps, mask=is_last)
                plsc.addupdate_scatter(iwsum, [ids], ps2 - bps2, mask=is_last)
            return 0

        lax.fori_loop(0, NVREG // 4, body, 0)

    pltpu.sync_copy(wsum, out_hbm.at[wid, 0])
    pltpu.sync_copy(iwsum, out_hbm.at[wid, 1])


def _sc_hist(keys, w, consts):
    mesh = plsc.VectorSubcoreMesh(core_axis_name="c", subcore_axis_name="s")
    return pl.kernel(
        _sc_hist_body,
        mesh=mesh,
        compiler_params=pltpu.CompilerParams(needs_layout_passes=False),
        out_type=jax.ShapeDtypeStruct((NTILES, 2, B), jnp.float32),
        scratch_types=[
            pltpu.VMEM((B,), jnp.float32),
            pltpu.VMEM((B,), jnp.float32),
            pltpu.VMEM((4, 16), jnp.float32),
            pltpu.VMEM((CHUNK,), jnp.float32),
            pltpu.VMEM((CHUNK,), jnp.float32),
            pltpu.VMEM((CHUNK,), jnp.float32),
            pltpu.VMEM((CHUNK,), jnp.float32),
            pltpu.SemaphoreType.DMA((2,)),
            pltpu.SemaphoreType.DMA((2,)),
        ],
    )(keys, w, consts)


def _final_body(hist_ref, h_ref, out_ref):
    a = hist_ref[...]
    ws = jnp.sum(a[:, 0, :], axis=0)
    iw = jnp.sum(a[:, 1, :], axis=0)
    h = h_ref[0, 0]

    ws2 = ws.reshape(256, 128)
    iw2 = iw.reshape(256, 128)

    ii = lax.broadcasted_iota(jnp.int32, (128, 128), 0)
    jj = lax.broadcasted_iota(jnp.int32, (128, 128), 1)
    ltri = jnp.where(ii <= jj, 1.0, 0.0).astype(jnp.float32)
    c1 = jax.lax.dot_general(ws2, ltri, (((1,), (0,)), ((), ())),
                             preferred_element_type=jnp.float32)
    rowtot = c1[:, 127:128]
    ri = lax.broadcasted_iota(jnp.int32, (256, 256), 0)
    rj = lax.broadcasted_iota(jnp.int32, (256, 256), 1)
    stri = jnp.where(ri > rj, 1.0, 0.0).astype(jnp.float32)
    rowoff = jax.lax.dot_general(stri, rowtot, (((1,), (0,)), ((), ())),
                                 preferred_element_type=jnp.float32)
    s_incl = c1 + rowoff
    s0 = s_incl - ws2
    out_ref[...] = jnp.sum(jnp.abs(s0 * h + iw2)).reshape(1, 1)


def _final_tc(hist, h11):
    return pl.pallas_call(
        _final_body,
        out_shape=jax.ShapeDtypeStruct((1, 1), jnp.float32),
    )(hist, h11)


def kernel(x, y, x_weights, y_weights):
    f32 = jnp.float32
    pad64 = jnp.zeros((64,), f32)
    xk2 = jnp.concatenate([x, jnp.full((64,), x[0], f32)]).reshape(7813, 128)
    yk2 = jnp.concatenate([y, jnp.full((64,), y[0], f32)]).reshape(7813, 128)
    xw2 = jnp.concatenate([x_weights, pad64]).reshape(7813, 128)
    yw2 = jnp.concatenate([y_weights, pad64]).reshape(7813, 128)

    vmin, vmax, wx, wy = _reduce_tc(xk2, yk2, xw2, yw2)
    vmin_s = vmin[0, 0]
    vmax_s = vmax[0, 0]
    h = jnp.maximum((vmax_s - vmin_s) / B, 1e-30)
    inv_h = 1.0 / h
    cx = 1.0 / wx[0, 0]
    cy = 1.0 / wy[0, 0]

    padk = jnp.zeros((PAD_SIDE,), f32)
    keys = jnp.concatenate([x, padk, y, padk])
    w = jnp.concatenate([x_weights, padk, -y_weights, padk])

    tile_c = jnp.where(jnp.arange(NTILES) < 16, cx, cy)
    consts = jnp.stack([
        jnp.full((NTILES,), vmin_s),
        jnp.full((NTILES,), h),
        jnp.full((NTILES,), inv_h),
        tile_c,
    ], axis=1)
    consts = jnp.broadcast_to(consts[:, :, None], (NTILES, 4, 16)) + 0.0

    hist = _sc_hist(keys, w, consts)

    out = _final_tc(hist, h.reshape(1, 1))
    return out[0, 0]


# speedup vs baseline: 1.6368x; 1.6368x over previous
import jax
import jax.numpy as jnp
from jax import lax
from jax.experimental import pallas as pl
from jax.experimental.pallas import tpu as pltpu
from jax.experimental.pallas import tpu_sc as plsc

NSAMP = 1000000
LANES = 16
NTILES = 32
CHUNK = 8000
FULLC = 3
REM_BASE = NTILES * FULLC * CHUNK
REM = (NSAMP - REM_BASE) // NTILES
REM_RD = 7256
B = 32768


def _reduce_body(xk, yk, xw, yw, vmin_o, vmax_o, wx_o, wy_o):
    vmin_o[...] = jnp.minimum(jnp.min(xk[...]), jnp.min(yk[...])).reshape(1, 1)
    vmax_o[...] = jnp.maximum(jnp.max(xk[...]), jnp.max(yk[...])).reshape(1, 1)
    wx_o[...] = jnp.sum(xw[...]).reshape(1, 1)
    wy_o[...] = jnp.sum(yw[...]).reshape(1, 1)


def _reduce_tc(x, y, xw, yw):
    s11 = jax.ShapeDtypeStruct((1, 1), jnp.float32)
    return pl.pallas_call(
        _reduce_body,
        out_shape=(s11, s11, s11, s11),
    )(x, y, xw, yw)


def _sc_hist_body(x_hbm, y_hbm, xw_hbm, yw_hbm, consts_hbm, out_hbm,
                  wsum, iwsum, cvm, kbuf0, kbuf1, wbuf0, wbuf1,
                  ksem, wsem):
    kbufs = (kbuf0, kbuf1)
    wbufs = (wbuf0, wbuf1)
    cid = lax.axis_index("c")
    sid = lax.axis_index("s")
    wid = sid * 2 + cid

    pltpu.sync_copy(consts_hbm.at[wid], cvm)
    vmin_v = cvm[0, :]
    h_v = cvm[1, :]
    inv_h_v = cvm[2, :]
    cx_v = cvm[3, :]
    cy_v = cvm[4, :]

    lane = lax.iota(jnp.int32, 16)
    lanem1 = jnp.maximum(lane - 1, 0)
    lanep1 = jnp.minimum(lane + 1, 15)
    zeros16 = jnp.zeros((16,), jnp.float32)

    def zero_body(i, _):
        for u in range(8):
            wsum[pl.ds(i * 128 + u * 16, 16)] = zeros16
            iwsum[pl.ds(i * 128 + u * 16, 16)] = zeros16
        return 0

    lax.fori_loop(0, B // 128, zero_body, 0)

    raw = wid * REM
    roff = pl.multiple_of(REM_BASE + (raw & ~7), 8)
    delta = raw & 7
    rem_lo = delta
    rem_hi = delta + REM

    sched = []
    for kr, wr, c_row in ((x_hbm, xw_hbm, cx_v), (y_hbm, yw_hbm, cy_v)):
        for i in range(FULLC):
            sched.append((kr, wr, c_row, pl.multiple_of((wid + NTILES * i) * CHUNK, 8),
                          CHUNK, False))
        sched.append((kr, wr, c_row, roff, REM_RD, True))

    def start_dma(idx, slot):
        kr, wr, _, off, size, _ = sched[idx]
        pltpu.async_copy(kr.at[pl.ds(off, size)], kbufs[slot].at[pl.ds(0, size)],
                         ksem.at[slot])
        pltpu.async_copy(wr.at[pl.ds(off, size)], wbufs[slot].at[pl.ds(0, size)],
                         wsem.at[slot])

    def wait_dma(idx, slot):
        size = sched[idx][4]
        pltpu.make_async_copy(x_hbm.at[pl.ds(0, size)],
                              kbufs[slot].at[pl.ds(0, size)],
                              ksem.at[slot]).wait()
        pltpu.make_async_copy(x_hbm.at[pl.ds(0, size)],
                              wbufs[slot].at[pl.ds(0, size)],
                              wsem.at[slot]).wait()

    def process(idx, slot):
        _, _, c_v, _, size, masked = sched[idx]
        kb = kbufs[slot]
        wb = wbufs[slot]
        nvg = (size + 63) // 64 if masked else size // 64

        def body(ii, _):
            for u in range(4):
                pos0 = ii * 64 + u * 16
                k = kb[pl.ds(pos0, 16)]
                w = wb[pl.ds(pos0, 16)]
                t = (k - vmin_v) * inv_h_v
                b = jnp.clip(t.astype(jnp.int32), 0, B - 1)
                s = w * c_v
                dd = (b.astype(jnp.float32) + 1.0) * h_v + vmin_v - k
                d = jnp.clip(dd, 0.0, h_v)
                if masked:
                    posv = pos0 + lane
                    m = (posv >= rem_lo) & (posv < rem_hi)
                    s = jnp.where(m, s, 0.0)
                    d = jnp.where(m, d, 0.0)
                ids, ss = plsc.sort_key_val(b, s)
                _, sds = plsc.sort_key_val(b, s * d)
                prev = ids.at[lanem1].get(mode="promise_in_bounds")
                nxt = ids.at[lanep1].get(mode="promise_in_bounds")
                newg = (lane == 0) | (ids != prev)
                is_last = (lane == 15) | (ids != nxt)
                gstart = plsc.cummax(jnp.where(newg, lane, 0))
                ps = plsc.cumsum(ss)
                ps2 = plsc.cumsum(sds)
                gm1 = jnp.maximum(gstart - 1, 0)
                first = gstart == 0
                bps = jnp.where(first, 0.0,
                                ps.at[gm1].get(mode="promise_in_bounds"))
                bps2 = jnp.where(first, 0.0,
                                 ps2.at[gm1].get(mode="promise_in_bounds"))
                plsc.addupdate_scatter(wsum, [ids], ps - bps, mask=is_last)
                plsc.addupdate_scatter(iwsum, [ids], ps2 - bps2,
                                       mask=is_last)
            return 0

        lax.fori_loop(0, nvg, body, 0)

    nsched = len(sched)
    start_dma(0, 0)
    for i in range(nsched):
        if i + 1 < nsched:
            start_dma(i + 1, (i + 1) % 2)
        process(i, i % 2)

    pltpu.sync_copy(wsum, out_hbm.at[wid, 0])
    pltpu.sync_copy(iwsum, out_hbm.at[wid, 1])


def _sc_hist(x, y, xw, yw, consts):
    mesh = plsc.VectorSubcoreMesh(core_axis_name="c", subcore_axis_name="s")
    return pl.kernel(
        _sc_hist_body,
        mesh=mesh,
        compiler_params=pltpu.CompilerParams(needs_layout_passes=False),
        out_type=jax.ShapeDtypeStruct((NTILES, 2, B), jnp.float32),
        scratch_types=[
            pltpu.VMEM((B,), jnp.float32),
            pltpu.VMEM((B,), jnp.float32),
            pltpu.VMEM((5, 16), jnp.float32),
            pltpu.VMEM((CHUNK,), jnp.float32),
            pltpu.VMEM((CHUNK,), jnp.float32),
            pltpu.VMEM((CHUNK,), jnp.float32),
            pltpu.VMEM((CHUNK,), jnp.float32),
            pltpu.SemaphoreType.DMA((2,)),
            pltpu.SemaphoreType.DMA((2,)),
        ],
    )(x, y, xw, yw, consts)


def _final_body(hist_ref, h_ref, out_ref):
    a = hist_ref[...]
    ws = jnp.sum(a[:, 0, :], axis=0)
    iw = jnp.sum(a[:, 1, :], axis=0)
    h = h_ref[0, 0]

    ws2 = ws.reshape(256, 128)
    iw2 = iw.reshape(256, 128)

    ii = lax.broadcasted_iota(jnp.int32, (128, 128), 0)
    jj = lax.broadcasted_iota(jnp.int32, (128, 128), 1)
    ltri = jnp.where(ii <= jj, 1.0, 0.0).astype(jnp.float32)
    c1 = jax.lax.dot_general(ws2, ltri, (((1,), (0,)), ((), ())),
                             preferred_element_type=jnp.float32)
    rowtot = c1[:, 127:128]
    ri = lax.broadcasted_iota(jnp.int32, (256, 256), 0)
    rj = lax.broadcasted_iota(jnp.int32, (256, 256), 1)
    stri = jnp.where(ri > rj, 1.0, 0.0).astype(jnp.float32)
    rowoff = jax.lax.dot_general(stri, rowtot, (((1,), (0,)), ((), ())),
                                 preferred_element_type=jnp.float32)
    s_incl = c1 + rowoff
    s0 = s_incl - ws2
    out_ref[...] = jnp.sum(jnp.abs(s0 * h + iw2)).reshape(1, 1)


def _final_tc(hist, h11):
    return pl.pallas_call(
        _final_body,
        out_shape=jax.ShapeDtypeStruct((1, 1), jnp.float32),
    )(hist, h11)


def kernel(x, y, x_weights, y_weights):
    vmin, vmax, wx, wy = _reduce_tc(x, y, x_weights, y_weights)
    vmin_s = vmin[0, 0]
    vmax_s = vmax[0, 0]
    h = jnp.maximum((vmax_s - vmin_s) / B, 1e-30)
    inv_h = 1.0 / h
    cx = 1.0 / wx[0, 0]
    cy = -1.0 / wy[0, 0]

    consts = jnp.stack([
        jnp.full((NTILES,), vmin_s),
        jnp.full((NTILES,), h),
        jnp.full((NTILES,), inv_h),
        jnp.full((NTILES,), cx),
        jnp.full((NTILES,), cy),
    ], axis=1)
    consts = jnp.broadcast_to(consts[:, :, None], (NTILES, 5, 16)) + 0.0

    hist = _sc_hist(x, y, x_weights, y_weights, consts)

    out = _final_tc(hist, h.reshape(1, 1))
    return out[0, 0]
